# baseline (speedup 1.0000x reference)
import functools
import math

import jax
from jax import lax
import jax.numpy as jnp
from jax.experimental import pallas as pl
from jax.experimental.pallas import tpu as pltpu
from jax.experimental.pallas import tpu_sc as plsc

T = 2048
D = 1024
E = 8
DFF = 1024
TOP_K = 2
_SCALE = 1.0 / math.sqrt(TOP_K)

BTS = 128
NBLK = 2 * T // BTS + E
SPAD = NBLK * BTS

NC, NS = 2, 16
NW = NC * NS

BT = 512
NT = T // BT


def _dot_t(a, b):
    return jax.lax.dot_general(a, b, (((1,), (1,)), ((), ())),
                               preferred_element_type=jnp.float32)


def _route_body(x_ref, gate_ref, sgw_ref, sgp_ref, sup_ref, sdn_ref,
                sa_ref, sb_ref, wa_ref, wb_ref, bexp_ref, gsh_ref):
    t = pl.program_id(0)
    xb = x_ref[pl.ds(t * BT, BT), :]
    gs = jax.nn.sigmoid(jnp.sum(xb * sgw_ref[...], axis=1, keepdims=True))
    g = _dot_t(xb, sgp_ref[...])
    u = _dot_t(xb, sup_ref[...])
    h = g * jax.nn.sigmoid(g) * u
    gsh_ref[...] = (gs * _SCALE) * _dot_t(h, sdn_ref[...])

    @pl.when(t == 0)
    def _route():
        _route_core(x_ref, gate_ref, sa_ref, sb_ref, wa_ref, wb_ref,
                    bexp_ref)


def _route_core(x_ref, gate_ref, sa_ref, sb_ref, wa_ref, wb_ref, bexp_ref):
    x = x_ref[...]
    logits = _dot_t(x, gate_ref[...])
    p = jax.nn.softmax(logits, axis=-1)
    m1 = jnp.max(p, axis=-1, keepdims=True)
    p_rest = jnp.where(p >= m1, -jnp.inf, p)
    m2 = jnp.max(p_rest, axis=-1, keepdims=True)
    mask = p >= m2
    pm = jnp.where(mask, p, 0.0)
    wd = pm / jnp.sum(pm, axis=-1, keepdims=True)

    ei = jax.lax.broadcasted_iota(jnp.int32, (T, E), 1)
    e1 = jnp.min(jnp.where(mask, ei, 8), axis=-1, keepdims=True)
    e2 = jnp.max(jnp.where(mask, ei, -1), axis=-1, keepdims=True)
    w1 = jnp.sum(jnp.where(ei == e1, wd, 0.0), axis=-1, keepdims=True)
    w2 = jnp.sum(jnp.where(ei == e2, wd, 0.0), axis=-1, keepdims=True)
    wa_ref[...] = (w1 * _SCALE) * jnp.ones((1, 16), jnp.float32)
    wb_ref[...] = (w2 * _SCALE) * jnp.ones((1, 16), jnp.float32)

    maskf = mask.astype(jnp.float32)
    s = maskf
    k = 1
    while k < T:
        s = s + jnp.concatenate([jnp.zeros((k, E), jnp.float32), s[:-k]],
                                axis=0)
        k *= 2
    rank = s - maskf
    count = jnp.sum(maskf, axis=0, keepdims=True)
    cpad = jnp.ceil(count * (1.0 / BTS)) * BTS
    base = jnp.zeros((1, 1), jnp.float32)
    bases = []
    for e in range(E):
        bases.append(base)
        base = base + cpad[:, e:e + 1]
    basev = jnp.concatenate(bases, axis=1)
    slotd = basev + rank
    sa = jnp.sum(jnp.where(ei == e1, slotd, 0.0), axis=-1, keepdims=True)
    sb = jnp.sum(jnp.where(ei == e2, slotd, 0.0), axis=-1, keepdims=True)
    sa_ref[...] = sa.astype(jnp.int32)
    sb_ref[...] = sb.astype(jnp.int32)

    ends = basev + cpad
    starts = jax.lax.broadcasted_iota(
        jnp.int32, (1, NBLK), 1).astype(jnp.float32) * BTS
    acc = jnp.zeros((1, NBLK), jnp.float32)
    for e in range(E):
        acc = acc + (starts >= ends[:, e:e + 1]).astype(jnp.float32)
    bexp_ref[...] = jnp.minimum(acc, float(E - 1)).astype(jnp.int32)


def _dispatch(x, idx2d):
    mesh = plsc.VectorSubcoreMesh(core_axis_name="c", subcore_axis_name="s")
    n_per_w = 2 * T // NW
    sub = 32
    nsub = n_per_w // sub

    @functools.partial(
        pl.kernel, mesh=mesh,
        out_type=jax.ShapeDtypeStruct((SPAD, D), jnp.float32),
        scratch_types=[pltpu.VMEM((nsub, sub), jnp.int32),
                       pltpu.VMEM((sub, D), jnp.float32),
                       pltpu.VMEM((sub, D), jnp.float32),
                       pltpu.VMEM((sub, D), jnp.float32),
                       pltpu.SemaphoreType.DMA,
                       pltpu.SemaphoreType.DMA,
                       pltpu.SemaphoreType.DMA,
                       pltpu.SemaphoreType.DMA,
                       pltpu.SemaphoreType.DMA,
                       pltpu.SemaphoreType.DMA],
    )
    def k(x_hbm, idx_hbm, xs_hbm, idx_v, rv0, rv1, rv2,
          ls0, ls1, ls2, ss0, ss1, ss2):
        wid = lax.axis_index("s") * NC + lax.axis_index("c")
        a0 = wid * n_per_w
        t0 = lax.rem(a0, T)
        pltpu.sync_copy(idx_hbm.at[pl.ds(wid * nsub, nsub)], idx_v)

        rvs, lss, sss = (rv0, rv1, rv2), (ls0, ls1, ls2), (ss0, ss1, ss2)
        loads = [pltpu.async_copy(x_hbm.at[pl.ds(t0 + s * sub, sub)],
                                  rvs[s], lss[s]) for s in range(3)]
        stores = {}
        for s in range(nsub):
            b = s % 3
            if s >= 3:
                stores[s - 3].wait()
                loads[b] = pltpu.async_copy(
                    x_hbm.at[pl.ds(t0 + s * sub, sub)], rvs[b], lss[b])
            loads[b].wait()
            stores[s] = pltpu.async_copy(rvs[b], xs_hbm.at[idx_v.at[s]],
                                         sss[b])
        for s in range(max(0, nsub - 3), nsub):
            stores[s].wait()

    return k(x, idx2d)


def _group_body(bexp_ref, xs_ref, wgp_ref, wup_ref, wdn_ref, ys_ref):
    del bexp_ref
    x = xs_ref[...]
    g = _dot_t(x, wgp_ref[0])
    u = _dot_t(x, wup_ref[0])
    h = g * jax.nn.sigmoid(g) * u
    ys_ref[...] = _dot_t(h, wdn_ref[0])


def _combine(ys, sa, sb, wa, wb, gsh):
    mesh = plsc.VectorSubcoreMesh(core_axis_name="c", subcore_axis_name="s")
    n_per_w = T // NW
    sub = 16
    nsub = n_per_w // sub

    @functools.partial(
        pl.kernel, mesh=mesh,
        out_type=jax.ShapeDtypeStruct((T, D), jnp.float32),
        scratch_types=[pltpu.VMEM((sub,), jnp.int32),
                       pltpu.VMEM((sub,), jnp.int32),
                       pltpu.VMEM((sub, 16), jnp.float32),
                       pltpu.VMEM((sub, 16), jnp.float32),
                       pltpu.VMEM((sub, D), jnp.float32),
                       pltpu.VMEM((sub, D), jnp.float32),
                       pltpu.VMEM((sub, D), jnp.float32),
                       pltpu.SemaphoreType.DMA,
                       pltpu.SemaphoreType.DMA,
                       pltpu.SemaphoreType.DMA],
    )
    def k(ys_hbm, sa_hbm, sb_hbm, wa_hbm, wb_hbm, gsh_hbm, out_hbm,
          ia_v, ib_v, wa_v, wb_v, ya_v, yb_v, o_v, sem_a, sem_b, sem_g):
        wid = lax.axis_index("s") * NC + lax.axis_index("c")
        t0 = wid * n_per_w

        @pl.loop(0, nsub)
        def _(s):
            rows = pl.ds(t0 + s * sub, sub)
            pltpu.sync_copy(sa_hbm.at[rows], ia_v)
            pltpu.sync_copy(sb_hbm.at[rows], ib_v)
            pltpu.sync_copy(wa_hbm.at[rows], wa_v)
            pltpu.sync_copy(wb_hbm.at[rows], wb_v)
            ca = pltpu.async_copy(ys_hbm.at[ia_v], ya_v, sem_a)
            cb = pltpu.async_copy(ys_hbm.at[ib_v], yb_v, sem_b)
            cg = pltpu.async_copy(gsh_hbm.at[rows], o_v, sem_g)
            ca.wait()
            cb.wait()
            cg.wait()

            @pl.loop(0, sub)
            def _(r):
                rr = pl.ds(r, 1)
                wav = wa_v.at[rr, :][...]
                wbv = wb_v.at[rr, :][...]

                @pl.loop(0, D // 16, step=4)
                def _(c):
                    for j in range(4):
                        cc = pl.ds((c + j) * 16, 16)
                        o_v.at[rr, cc][...] += (
                            wav * ya_v.at[rr, cc][...]
                            + wbv * yb_v.at[rr, cc][...])

            pltpu.sync_copy(o_v, out_hbm.at[rows])

    return k(ys, sa, sb, wa, wb, gsh)


@jax.jit
def kernel(hidden_states, gate_w, shared_gate_w, Wgp, Wup, Wdn, Sgp, Sup, Sdn):
    x = hidden_states.reshape(T, D)

    full = lambda s: pl.BlockSpec(s, lambda *_: (0,) * len(s))
    tok = lambda d1: pl.BlockSpec((BT, d1), lambda t: (t, 0))

    sa, sb, wa, wb, bexp, gsh = pl.pallas_call(
        _route_body,
        grid=(NT,),
        in_specs=[full((T, D)), full((E, D)), full((1, D)),
                  full((DFF, D)), full((DFF, D)), full((D, DFF))],
        out_specs=(full((T, 1)), full((T, 1)), full((T, 16)),
                   full((T, 16)), full((1, NBLK)), tok(D)),
        out_shape=(jax.ShapeDtypeStruct((T, 1), jnp.int32),
                   jax.ShapeDtypeStruct((T, 1), jnp.int32),
                   jax.ShapeDtypeStruct((T, 16), jnp.float32),
                   jax.ShapeDtypeStruct((T, 16), jnp.float32),
                   jax.ShapeDtypeStruct((1, NBLK), jnp.int32),
                   jax.ShapeDtypeStruct((T, D), jnp.float32)),
    )(x, gate_w, shared_gate_w, Sgp, Sup, Sdn)

    idx = jnp.concatenate([sa.reshape(T), sb.reshape(T)]).reshape(-1, 32)
    xs = _dispatch(x, idx)

    ys = pl.pallas_call(
        _group_body,
        grid_spec=pltpu.PrefetchScalarGridSpec(
            num_scalar_prefetch=1,
            grid=(NBLK,),
            in_specs=[
                pl.BlockSpec((BTS, D), lambda i, bexp_ref: (i, 0)),
                pl.BlockSpec((1, DFF, D),
                             lambda i, bexp_ref: (bexp_ref[i], 0, 0)),
                pl.BlockSpec((1, DFF, D),
                             lambda i, bexp_ref: (bexp_ref[i], 0, 0)),
                pl.BlockSpec((1, D, DFF),
                             lambda i, bexp_ref: (bexp_ref[i], 0, 0)),
            ],
            out_specs=pl.BlockSpec((BTS, D), lambda i, bexp_ref: (i, 0)),
        ),
        out_shape=jax.ShapeDtypeStruct((SPAD, D), jnp.float32),
    )(bexp.reshape(NBLK), xs, Wgp, Wup, Wdn)

    return _combine(ys, sa.reshape(T), sb.reshape(T), wa, wb, gsh)


# speedup vs baseline: 1.2500x; 1.2500x over previous
import functools
import math

import jax
from jax import lax
import jax.numpy as jnp
from jax.experimental import pallas as pl
from jax.experimental.pallas import tpu as pltpu
from jax.experimental.pallas import tpu_sc as plsc

T = 2048
D = 1024
E = 8
DFF = 1024
TOP_K = 2
_SCALE = 1.0 / math.sqrt(TOP_K)

BTS = 256
NBLK = 2 * T // BTS + E
SPAD = NBLK * BTS

NC, NS = 2, 16
NW = NC * NS

BT = 512
NT = T // BT


def _dot_t(a, b):
    return jax.lax.dot_general(a, b, (((1,), (1,)), ((), ())),
                               preferred_element_type=jnp.float32)


def _route_body(x_ref, gate_ref, sgw_ref, sgp_ref, sup_ref, sdn_ref,
                sa_ref, sb_ref, wa_ref, wb_ref, bexp_ref, gsh_ref):
    t = pl.program_id(0)
    xb = x_ref[pl.ds(t * BT, BT), :]
    gs = jax.nn.sigmoid(jnp.sum(xb * sgw_ref[...], axis=1, keepdims=True))
    g = _dot_t(xb, sgp_ref[...])
    u = _dot_t(xb, sup_ref[...])
    h = g * jax.nn.sigmoid(g) * u
    gsh_ref[...] = (gs * _SCALE) * _dot_t(h, sdn_ref[...])

    @pl.when(t == 0)
    def _route():
        _route_core(x_ref, gate_ref, sa_ref, sb_ref, wa_ref, wb_ref,
                    bexp_ref)


def _route_core(x_ref, gate_ref, sa_ref, sb_ref, wa_ref, wb_ref, bexp_ref):
    x = x_ref[...]
    logits = _dot_t(x, gate_ref[...])
    p = jax.nn.softmax(logits, axis=-1)
    m1 = jnp.max(p, axis=-1, keepdims=True)
    p_rest = jnp.where(p >= m1, -jnp.inf, p)
    m2 = jnp.max(p_rest, axis=-1, keepdims=True)
    mask = p >= m2
    pm = jnp.where(mask, p, 0.0)
    wd = pm / jnp.sum(pm, axis=-1, keepdims=True)

    ei = jax.lax.broadcasted_iota(jnp.int32, (T, E), 1)
    e1 = jnp.min(jnp.where(mask, ei, 8), axis=-1, keepdims=True)
    e2 = jnp.max(jnp.where(mask, ei, -1), axis=-1, keepdims=True)
    w1 = jnp.sum(jnp.where(ei == e1, wd, 0.0), axis=-1, keepdims=True)
    w2 = jnp.sum(jnp.where(ei == e2, wd, 0.0), axis=-1, keepdims=True)
    wa_ref[...] = (w1 * _SCALE) * jnp.ones((1, 16), jnp.float32)
    wb_ref[...] = (w2 * _SCALE) * jnp.ones((1, 16), jnp.float32)

    maskf = mask.astype(jnp.float32)
    s = maskf
    k = 1
    while k < T:
        s = s + jnp.concatenate([jnp.zeros((k, E), jnp.float32), s[:-k]],
                                axis=0)
        k *= 2
    rank = s - maskf
    count = jnp.sum(maskf, axis=0, keepdims=True)
    cpad = jnp.ceil(count * (1.0 / BTS)) * BTS
    base = jnp.zeros((1, 1), jnp.float32)
    bases = []
    for e in range(E):
        bases.append(base)
        base = base + cpad[:, e:e + 1]
    basev = jnp.concatenate(bases, axis=1)
    slotd = basev + rank
    sa = jnp.sum(jnp.where(ei == e1, slotd, 0.0), axis=-1, keepdims=True)
    sb = jnp.sum(jnp.where(ei == e2, slotd, 0.0), axis=-1, keepdims=True)
    sa_ref[...] = sa.astype(jnp.int32)
    sb_ref[...] = sb.astype(jnp.int32)

    ends = basev + cpad
    starts = jax.lax.broadcasted_iota(
        jnp.int32, (1, NBLK), 1).astype(jnp.float32) * BTS
    acc = jnp.zeros((1, NBLK), jnp.float32)
    for e in range(E):
        acc = acc + (starts >= ends[:, e:e + 1]).astype(jnp.float32)
    bexp_ref[...] = jnp.minimum(acc, float(E - 1)).astype(jnp.int32)


def _dispatch(x, idx2d):
    mesh = plsc.VectorSubcoreMesh(core_axis_name="c", subcore_axis_name="s")
    n_per_w = 2 * T // NW
    sub = 32
    nsub = n_per_w // sub

    @functools.partial(
        pl.kernel, mesh=mesh,
        out_type=jax.ShapeDtypeStruct((SPAD, D), jnp.float32),
        scratch_types=[pltpu.VMEM((nsub, sub), jnp.int32),
                       pltpu.VMEM((sub, D), jnp.float32),
                       pltpu.VMEM((sub, D), jnp.float32),
                       pltpu.VMEM((sub, D), jnp.float32),
                       pltpu.SemaphoreType.DMA,
                       pltpu.SemaphoreType.DMA,
                       pltpu.SemaphoreType.DMA,
                       pltpu.SemaphoreType.DMA,
                       pltpu.SemaphoreType.DMA,
                       pltpu.SemaphoreType.DMA],
    )
    def k(x_hbm, idx_hbm, xs_hbm, idx_v, rv0, rv1, rv2,
          ls0, ls1, ls2, ss0, ss1, ss2):
        wid = lax.axis_index("s") * NC + lax.axis_index("c")
        a0 = wid * n_per_w
        t0 = lax.rem(a0, T)
        pltpu.sync_copy(idx_hbm.at[pl.ds(wid * nsub, nsub)], idx_v)

        rvs, lss, sss = (rv0, rv1, rv2), (ls0, ls1, ls2), (ss0, ss1, ss2)
        loads = [pltpu.async_copy(x_hbm.at[pl.ds(t0 + s * sub, sub)],
                                  rvs[s], lss[s]) for s in range(3)]
        stores = {}
        for s in range(nsub):
            b = s % 3
            if s >= 3:
                stores[s - 3].wait()
                loads[b] = pltpu.async_copy(
                    x_hbm.at[pl.ds(t0 + s * sub, sub)], rvs[b], lss[b])
            loads[b].wait()
            stores[s] = pltpu.async_copy(rvs[b], xs_hbm.at[idx_v.at[s]],
                                         sss[b])
        for s in range(max(0, nsub - 3), nsub):
            stores[s].wait()

    return k(x, idx2d)


def _group_body(bexp_ref, xs_ref, wgp_ref, wup_ref, wdn_ref, ys_ref):
    del bexp_ref
    x = xs_ref[...]
    g = _dot_t(x, wgp_ref[0])
    u = _dot_t(x, wup_ref[0])
    h = g * jax.nn.sigmoid(g) * u
    ys_ref[...] = _dot_t(h, wdn_ref[0])


def _combine(ys, sa, sb, wa, wb, gsh):
    mesh = plsc.VectorSubcoreMesh(core_axis_name="c", subcore_axis_name="s")
    n_per_w = T // NW
    sub = 16
    nsub = n_per_w // sub

    @functools.partial(
        pl.kernel, mesh=mesh,
        out_type=jax.ShapeDtypeStruct((T, D), jnp.float32),
        scratch_types=[pltpu.VMEM((sub,), jnp.int32),
                       pltpu.VMEM((sub,), jnp.int32),
                       pltpu.VMEM((sub, 16), jnp.float32),
                       pltpu.VMEM((sub, 16), jnp.float32),
                       pltpu.VMEM((sub, D), jnp.float32),
                       pltpu.VMEM((sub, D), jnp.float32),
                       pltpu.VMEM((sub, D), jnp.float32),
                       pltpu.SemaphoreType.DMA,
                       pltpu.SemaphoreType.DMA,
                       pltpu.SemaphoreType.DMA],
    )
    def k(ys_hbm, sa_hbm, sb_hbm, wa_hbm, wb_hbm, gsh_hbm, out_hbm,
          ia_v, ib_v, wa_v, wb_v, ya_v, yb_v, o_v, sem_a, sem_b, sem_g):
        wid = lax.axis_index("s") * NC + lax.axis_index("c")
        t0 = wid * n_per_w

        @pl.loop(0, nsub)
        def _(s):
            rows = pl.ds(t0 + s * sub, sub)
            pltpu.sync_copy(sa_hbm.at[rows], ia_v)
            pltpu.sync_copy(sb_hbm.at[rows], ib_v)
            pltpu.sync_copy(wa_hbm.at[rows], wa_v)
            pltpu.sync_copy(wb_hbm.at[rows], wb_v)
            ca = pltpu.async_copy(ys_hbm.at[ia_v], ya_v, sem_a)
            cb = pltpu.async_copy(ys_hbm.at[ib_v], yb_v, sem_b)
            cg = pltpu.async_copy(gsh_hbm.at[rows], o_v, sem_g)
            ca.wait()
            cb.wait()
            cg.wait()

            @pl.loop(0, sub)
            def _(r):
                rr = pl.ds(r, 1)
                wav = wa_v.at[rr, :][...]
                wbv = wb_v.at[rr, :][...]

                @pl.loop(0, D // 16, step=4)
                def _(c):
                    for j in range(4):
                        cc = pl.ds((c + j) * 16, 16)
                        o_v.at[rr, cc][...] += (
                            wav * ya_v.at[rr, cc][...]
                            + wbv * yb_v.at[rr, cc][...])

            pltpu.sync_copy(o_v, out_hbm.at[rows])

    return k(ys, sa, sb, wa, wb, gsh)


@jax.jit
def kernel(hidden_states, gate_w, shared_gate_w, Wgp, Wup, Wdn, Sgp, Sup, Sdn):
    x = hidden_states.reshape(T, D)

    full = lambda s: pl.BlockSpec(s, lambda *_: (0,) * len(s))
    tok = lambda d1: pl.BlockSpec((BT, d1), lambda t: (t, 0))

    sa, sb, wa, wb, bexp, gsh = pl.pallas_call(
        _route_body,
        grid=(NT,),
        in_specs=[full((T, D)), full((E, D)), full((1, D)),
                  full((DFF, D)), full((DFF, D)), full((D, DFF))],
        out_specs=(full((T, 1)), full((T, 1)), full((T, 16)),
                   full((T, 16)), full((1, NBLK)), tok(D)),
        out_shape=(jax.ShapeDtypeStruct((T, 1), jnp.int32),
                   jax.ShapeDtypeStruct((T, 1), jnp.int32),
                   jax.ShapeDtypeStruct((T, 16), jnp.float32),
                   jax.ShapeDtypeStruct((T, 16), jnp.float32),
                   jax.ShapeDtypeStruct((1, NBLK), jnp.int32),
                   jax.ShapeDtypeStruct((T, D), jnp.float32)),
    )(x, gate_w, shared_gate_w, Sgp, Sup, Sdn)

    idx = jnp.concatenate([sa.reshape(T), sb.reshape(T)]).reshape(-1, 32)
    xs = _dispatch(x, idx)

    ys = pl.pallas_call(
        _group_body,
        grid_spec=pltpu.PrefetchScalarGridSpec(
            num_scalar_prefetch=1,
            grid=(NBLK,),
            in_specs=[
                pl.BlockSpec((BTS, D), lambda i, bexp_ref: (i, 0)),
                pl.BlockSpec((1, DFF, D),
                             lambda i, bexp_ref: (bexp_ref[i], 0, 0)),
                pl.BlockSpec((1, DFF, D),
                             lambda i, bexp_ref: (bexp_ref[i], 0, 0)),
                pl.BlockSpec((1, D, DFF),
                             lambda i, bexp_ref: (bexp_ref[i], 0, 0)),
            ],
            out_specs=pl.BlockSpec((BTS, D), lambda i, bexp_ref: (i, 0)),
        ),
        out_shape=jax.ShapeDtypeStruct((SPAD, D), jnp.float32),
    )(bexp.reshape(NBLK), xs, Wgp, Wup, Wdn)

    return _combine(ys, sa.reshape(T), sb.reshape(T), wa, wb, gsh)


# speedup vs baseline: 1.3290x; 1.0631x over previous
import functools
import math

import jax
from jax import lax
import jax.numpy as jnp
from jax.experimental import pallas as pl
from jax.experimental.pallas import tpu as pltpu
from jax.experimental.pallas import tpu_sc as plsc

T = 2048
D = 1024
E = 8
DFF = 1024
TOP_K = 2
_SCALE = 1.0 / math.sqrt(TOP_K)

BTS = 256
NBLK = 2 * T // BTS + E
SPAD = NBLK * BTS

NC, NS = 2, 16
NW = NC * NS

BT = 512
NT = T // BT


def _dot_t(a, b):
    return jax.lax.dot_general(a, b, (((1,), (1,)), ((), ())),
                               preferred_element_type=jnp.float32)


def _route_body(x_ref, gate_ref, sa_ref, sb_ref, wa_ref, wb_ref, bexp_ref):
    x = x_ref[...]
    logits = _dot_t(x, gate_ref[...])
    p = jax.nn.softmax(logits, axis=-1)
    m1 = jnp.max(p, axis=-1, keepdims=True)
    p_rest = jnp.where(p >= m1, -jnp.inf, p)
    m2 = jnp.max(p_rest, axis=-1, keepdims=True)
    mask = p >= m2
    pm = jnp.where(mask, p, 0.0)
    wd = pm / jnp.sum(pm, axis=-1, keepdims=True)

    ei = jax.lax.broadcasted_iota(jnp.int32, (T, E), 1)
    e1 = jnp.min(jnp.where(mask, ei, 8), axis=-1, keepdims=True)
    e2 = jnp.max(jnp.where(mask, ei, -1), axis=-1, keepdims=True)
    w1 = jnp.sum(jnp.where(ei == e1, wd, 0.0), axis=-1, keepdims=True)
    w2 = jnp.sum(jnp.where(ei == e2, wd, 0.0), axis=-1, keepdims=True)
    wa_ref[...] = (w1 * _SCALE) * jnp.ones((1, 16), jnp.float32)
    wb_ref[...] = (w2 * _SCALE) * jnp.ones((1, 16), jnp.float32)

    maskf = mask.astype(jnp.float32)
    s = maskf
    k = 1
    while k < T:
        s = s + jnp.concatenate([jnp.zeros((k, E), jnp.float32), s[:-k]],
                                axis=0)
        k *= 2
    rank = s - maskf
    count = jnp.sum(maskf, axis=0, keepdims=True)
    cpad = jnp.ceil(count * (1.0 / BTS)) * BTS
    base = jnp.zeros((1, 1), jnp.float32)
    bases = []
    for e in range(E):
        bases.append(base)
        base = base + cpad[:, e:e + 1]
    basev = jnp.concatenate(bases, axis=1)
    slotd = basev + rank
    sa = jnp.sum(jnp.where(ei == e1, slotd, 0.0), axis=-1, keepdims=True)
    sb = jnp.sum(jnp.where(ei == e2, slotd, 0.0), axis=-1, keepdims=True)
    sa_ref[...] = sa.astype(jnp.int32)
    sb_ref[...] = sb.astype(jnp.int32)

    ends = basev + cpad
    starts = jax.lax.broadcasted_iota(
        jnp.int32, (1, NBLK), 1).astype(jnp.float32) * BTS
    acc = jnp.zeros((1, NBLK), jnp.float32)
    for e in range(E):
        acc = acc + (starts >= ends[:, e:e + 1]).astype(jnp.float32)
    bexp_ref[...] = jnp.minimum(acc, float(E - 1)).astype(jnp.int32)


def _shared_body(x_ref, sgw_ref, sgp_ref, sup_ref, sdn_ref, gsh_ref):
    x = x_ref[...]
    gs = jax.nn.sigmoid(jnp.sum(x * sgw_ref[...], axis=1, keepdims=True))
    g = _dot_t(x, sgp_ref[...])
    u = _dot_t(x, sup_ref[...])
    h = g * jax.nn.sigmoid(g) * u
    gsh_ref[...] = (gs * _SCALE) * _dot_t(h, sdn_ref[...])


def _dispatch(x, idx2d):
    mesh = plsc.VectorSubcoreMesh(core_axis_name="c", subcore_axis_name="s")
    n_per_w = 2 * T // NW
    sub = 32
    nsub = n_per_w // sub

    @functools.partial(
        pl.kernel, mesh=mesh,
        out_type=jax.ShapeDtypeStruct((SPAD, D), jnp.float32),
        scratch_types=[pltpu.VMEM((nsub, sub), jnp.int32),
                       pltpu.VMEM((sub, D), jnp.float32),
                       pltpu.VMEM((sub, D), jnp.float32),
                       pltpu.VMEM((sub, D), jnp.float32),
                       pltpu.SemaphoreType.DMA,
                       pltpu.SemaphoreType.DMA,
                       pltpu.SemaphoreType.DMA,
                       pltpu.SemaphoreType.DMA,
                       pltpu.SemaphoreType.DMA,
                       pltpu.SemaphoreType.DMA],
    )
    def k(x_hbm, idx_hbm, xs_hbm, idx_v, rv0, rv1, rv2,
          ls0, ls1, ls2, ss0, ss1, ss2):
        wid = lax.axis_index("s") * NC + lax.axis_index("c")
        a0 = wid * n_per_w
        t0 = lax.rem(a0, T)
        pltpu.sync_copy(idx_hbm.at[pl.ds(wid * nsub, nsub)], idx_v)

        rvs, lss, sss = (rv0, rv1, rv2), (ls0, ls1, ls2), (ss0, ss1, ss2)
        loads = [pltpu.async_copy(x_hbm.at[pl.ds(t0 + s * sub, sub)],
                                  rvs[s], lss[s]) for s in range(3)]
        stores = {}
        for s in range(nsub):
            b = s % 3
            if s >= 3:
                stores[s - 3].wait()
                loads[b] = pltpu.async_copy(
                    x_hbm.at[pl.ds(t0 + s * sub, sub)], rvs[b], lss[b])
            loads[b].wait()
            stores[s] = pltpu.async_copy(rvs[b], xs_hbm.at[idx_v.at[s]],
                                         sss[b])
        for s in range(max(0, nsub - 3), nsub):
            stores[s].wait()

    return k(x, idx2d)


def _group_body(bexp_ref, xs_ref, wgp_ref, wup_ref, wdn_ref, ys_ref):
    del bexp_ref
    x = xs_ref[...]
    g = _dot_t(x, wgp_ref[0])
    u = _dot_t(x, wup_ref[0])
    h = g * jax.nn.sigmoid(g) * u
    ys_ref[...] = _dot_t(h, wdn_ref[0])


def _combine(ys, sa, sb, wa, wb, gsh):
    mesh = plsc.VectorSubcoreMesh(core_axis_name="c", subcore_axis_name="s")
    n_per_w = T // NW
    sub = 16
    nsub = n_per_w // sub

    @functools.partial(
        pl.kernel, mesh=mesh,
        out_type=jax.ShapeDtypeStruct((T, D), jnp.float32),
        scratch_types=[pltpu.VMEM((sub,), jnp.int32),
                       pltpu.VMEM((sub,), jnp.int32),
                       pltpu.VMEM((sub, 16), jnp.float32),
                       pltpu.VMEM((sub, 16), jnp.float32),
                       pltpu.VMEM((sub, D), jnp.float32),
                       pltpu.VMEM((sub, D), jnp.float32),
                       pltpu.VMEM((sub, D), jnp.float32),
                       pltpu.SemaphoreType.DMA,
                       pltpu.SemaphoreType.DMA,
                       pltpu.SemaphoreType.DMA],
    )
    def k(ys_hbm, sa_hbm, sb_hbm, wa_hbm, wb_hbm, gsh_hbm, out_hbm,
          ia_v, ib_v, wa_v, wb_v, ya_v, yb_v, o_v, sem_a, sem_b, sem_g):
        wid = lax.axis_index("s") * NC + lax.axis_index("c")
        t0 = wid * n_per_w

        @pl.loop(0, nsub)
        def _(s):
            rows = pl.ds(t0 + s * sub, sub)
            pltpu.sync_copy(sa_hbm.at[rows], ia_v)
            pltpu.sync_copy(sb_hbm.at[rows], ib_v)
            pltpu.sync_copy(wa_hbm.at[rows], wa_v)
            pltpu.sync_copy(wb_hbm.at[rows], wb_v)
            ca = pltpu.async_copy(ys_hbm.at[ia_v], ya_v, sem_a)
            cb = pltpu.async_copy(ys_hbm.at[ib_v], yb_v, sem_b)
            cg = pltpu.async_copy(gsh_hbm.at[rows], o_v, sem_g)
            ca.wait()
            cb.wait()
            cg.wait()

            @pl.loop(0, sub)
            def _(r):
                rr = pl.ds(r, 1)
                wav = wa_v.at[rr, :][...]
                wbv = wb_v.at[rr, :][...]

                @pl.loop(0, D // 16, step=4)
                def _(c):
                    for j in range(4):
                        cc = pl.ds((c + j) * 16, 16)
                        o_v.at[rr, cc][...] += (
                            wav * ya_v.at[rr, cc][...]
                            + wbv * yb_v.at[rr, cc][...])

            pltpu.sync_copy(o_v, out_hbm.at[rows])

    return k(ys, sa, sb, wa, wb, gsh)


@jax.jit
def kernel(hidden_states, gate_w, shared_gate_w, Wgp, Wup, Wdn, Sgp, Sup, Sdn):
    x = hidden_states.reshape(T, D)

    full = lambda s: pl.BlockSpec(s, lambda *_: (0,) * len(s))
    tok = lambda d1: pl.BlockSpec((BT, d1), lambda t: (t, 0))

    sa, sb, wa, wb, bexp = pl.pallas_call(
        _route_body,
        in_specs=[full((T, D)), full((E, D))],
        out_specs=(full((T, 1)), full((T, 1)), full((T, 16)),
                   full((T, 16)), full((1, NBLK))),
        out_shape=(jax.ShapeDtypeStruct((T, 1), jnp.int32),
                   jax.ShapeDtypeStruct((T, 1), jnp.int32),
                   jax.ShapeDtypeStruct((T, 16), jnp.float32),
                   jax.ShapeDtypeStruct((T, 16), jnp.float32),
                   jax.ShapeDtypeStruct((1, NBLK), jnp.int32)),
    )(x, gate_w)

    gsh = pl.pallas_call(
        _shared_body,
        grid=(NT,),
        in_specs=[tok(D), full((1, D)), full((DFF, D)), full((DFF, D)),
                  full((D, DFF))],
        out_specs=tok(D),
        out_shape=jax.ShapeDtypeStruct((T, D), jnp.float32),
    )(x, shared_gate_w, Sgp, Sup, Sdn)

    idx = jnp.concatenate([sa.reshape(T), sb.reshape(T)]).reshape(-1, 32)
    xs = _dispatch(x, idx)

    ys = pl.pallas_call(
        _group_body,
        grid_spec=pltpu.PrefetchScalarGridSpec(
            num_scalar_prefetch=1,
            grid=(NBLK,),
            in_specs=[
                pl.BlockSpec((BTS, D), lambda i, bexp_ref: (i, 0)),
                pl.BlockSpec((1, DFF, D),
                             lambda i, bexp_ref: (bexp_ref[i], 0, 0)),
                pl.BlockSpec((1, DFF, D),
                             lambda i, bexp_ref: (bexp_ref[i], 0, 0)),
                pl.BlockSpec((1, D, DFF),
                             lambda i, bexp_ref: (bexp_ref[i], 0, 0)),
            ],
            out_specs=pl.BlockSpec((BTS, D), lambda i, bexp_ref: (i, 0)),
        ),
        out_shape=jax.ShapeDtypeStruct((SPAD, D), jnp.float32),
    )(bexp.reshape(NBLK), xs, Wgp, Wup, Wdn)

    return _combine(ys, sa.reshape(T), sb.reshape(T), wa, wb, gsh)


# speedup vs baseline: 1.3819x; 1.0398x over previous
import functools
import math

import jax
from jax import lax
import jax.numpy as jnp
from jax.experimental import pallas as pl
from jax.experimental.pallas import tpu as pltpu
from jax.experimental.pallas import tpu_sc as plsc

T = 2048
D = 1024
E = 8
DFF = 1024
TOP_K = 2
_SCALE = 1.0 / math.sqrt(TOP_K)

BTS = 256
NBLK = 2 * T // BTS + E
SPAD = NBLK * BTS

NC, NS = 2, 16
NW = NC * NS

BT = 512
NT = T // BT


def _dot_t(a, b):
    return jax.lax.dot_general(a, b, (((1,), (1,)), ((), ())),
                               preferred_element_type=jnp.float32)


def _route_body(x_ref, gate_ref, sa_ref, sb_ref, wa_ref, wb_ref, bexp_ref):
    x = x_ref[...]
    logits = _dot_t(x, gate_ref[...])
    p = jax.nn.softmax(logits, axis=-1)
    m1 = jnp.max(p, axis=-1, keepdims=True)
    p_rest = jnp.where(p >= m1, -jnp.inf, p)
    m2 = jnp.max(p_rest, axis=-1, keepdims=True)
    mask = p >= m2
    pm = jnp.where(mask, p, 0.0)
    wd = pm / jnp.sum(pm, axis=-1, keepdims=True)

    ei = jax.lax.broadcasted_iota(jnp.int32, (T, E), 1)
    e1 = jnp.min(jnp.where(mask, ei, 8), axis=-1, keepdims=True)
    e2 = jnp.max(jnp.where(mask, ei, -1), axis=-1, keepdims=True)
    w1 = jnp.sum(jnp.where(ei == e1, wd, 0.0), axis=-1, keepdims=True)
    w2 = jnp.sum(jnp.where(ei == e2, wd, 0.0), axis=-1, keepdims=True)
    wa_ref[...] = (w1 * _SCALE) * jnp.ones((1, 16), jnp.float32)
    wb_ref[...] = (w2 * _SCALE) * jnp.ones((1, 16), jnp.float32)

    maskf = mask.astype(jnp.float32)
    s = maskf
    k = 1
    while k < T:
        s = s + jnp.concatenate([jnp.zeros((k, E), jnp.float32), s[:-k]],
                                axis=0)
        k *= 2
    rank = s - maskf
    count = jnp.sum(maskf, axis=0, keepdims=True)
    cpad = jnp.ceil(count * (1.0 / BTS)) * BTS
    base = jnp.zeros((1, 1), jnp.float32)
    bases = []
    for e in range(E):
        bases.append(base)
        base = base + cpad[:, e:e + 1]
    basev = jnp.concatenate(bases, axis=1)
    slotd = basev + rank
    sa = jnp.sum(jnp.where(ei == e1, slotd, 0.0), axis=-1, keepdims=True)
    sb = jnp.sum(jnp.where(ei == e2, slotd, 0.0), axis=-1, keepdims=True)
    sa_ref[...] = sa.astype(jnp.int32)
    sb_ref[...] = sb.astype(jnp.int32)

    ends = basev + cpad
    starts = jax.lax.broadcasted_iota(
        jnp.int32, (1, NBLK), 1).astype(jnp.float32) * BTS
    acc = jnp.zeros((1, NBLK), jnp.float32)
    for e in range(E):
        acc = acc + (starts >= ends[:, e:e + 1]).astype(jnp.float32)
    bexp_ref[...] = acc.astype(jnp.int32)


def _shared_body(x_ref, sgw_ref, sgp_ref, sup_ref, sdn_ref, gsh_ref):
    x = x_ref[...]
    gs = jax.nn.sigmoid(jnp.sum(x * sgw_ref[...], axis=1, keepdims=True))
    g = _dot_t(x, sgp_ref[...])
    u = _dot_t(x, sup_ref[...])
    h = g * jax.nn.sigmoid(g) * u
    gsh_ref[...] = (gs * _SCALE) * _dot_t(h, sdn_ref[...])


def _dispatch(x, idx2d):
    mesh = plsc.VectorSubcoreMesh(core_axis_name="c", subcore_axis_name="s")
    n_per_w = 2 * T // NW
    sub = 32
    nsub = n_per_w // sub

    @functools.partial(
        pl.kernel, mesh=mesh,
        out_type=jax.ShapeDtypeStruct((SPAD, D), jnp.float32),
        scratch_types=[pltpu.VMEM((nsub, sub), jnp.int32),
                       pltpu.VMEM((sub, D), jnp.float32),
                       pltpu.VMEM((sub, D), jnp.float32),
                       pltpu.VMEM((sub, D), jnp.float32),
                       pltpu.SemaphoreType.DMA,
                       pltpu.SemaphoreType.DMA,
                       pltpu.SemaphoreType.DMA,
                       pltpu.SemaphoreType.DMA,
                       pltpu.SemaphoreType.DMA,
                       pltpu.SemaphoreType.DMA],
    )
    def k(x_hbm, idx_hbm, xs_hbm, idx_v, rv0, rv1, rv2,
          ls0, ls1, ls2, ss0, ss1, ss2):
        wid = lax.axis_index("s") * NC + lax.axis_index("c")
        a0 = wid * n_per_w
        t0 = lax.rem(a0, T)
        pltpu.sync_copy(idx_hbm.at[pl.ds(wid * nsub, nsub)], idx_v)

        rvs, lss, sss = (rv0, rv1, rv2), (ls0, ls1, ls2), (ss0, ss1, ss2)
        loads = [pltpu.async_copy(x_hbm.at[pl.ds(t0 + s * sub, sub)],
                                  rvs[s], lss[s]) for s in range(3)]
        stores = {}
        for s in range(nsub):
            b = s % 3
            if s >= 3:
                stores[s - 3].wait()
                loads[b] = pltpu.async_copy(
                    x_hbm.at[pl.ds(t0 + s * sub, sub)], rvs[b], lss[b])
            loads[b].wait()
            stores[s] = pltpu.async_copy(rvs[b], xs_hbm.at[idx_v.at[s]],
                                         sss[b])
        for s in range(max(0, nsub - 3), nsub):
            stores[s].wait()

    return k(x, idx2d)


def _group_body(bexp_ref, xs_ref, wgp_ref, wup_ref, wdn_ref, ys_ref):
    i = pl.program_id(0)

    @pl.when(bexp_ref[i] < E)
    def _():
        x = xs_ref[...]
        g = _dot_t(x, wgp_ref[0])
        u = _dot_t(x, wup_ref[0])
        h = g * jax.nn.sigmoid(g) * u
        ys_ref[...] = _dot_t(h, wdn_ref[0])


def _combine(ys, sa, sb, wa, wb, gsh):
    mesh = plsc.VectorSubcoreMesh(core_axis_name="c", subcore_axis_name="s")
    n_per_w = T // NW
    sub = 16
    nsub = n_per_w // sub

    @functools.partial(
        pl.kernel, mesh=mesh,
        out_type=jax.ShapeDtypeStruct((T, D), jnp.float32),
        scratch_types=[pltpu.VMEM((sub,), jnp.int32),
                       pltpu.VMEM((sub,), jnp.int32),
                       pltpu.VMEM((sub, 16), jnp.float32),
                       pltpu.VMEM((sub, 16), jnp.float32),
                       pltpu.VMEM((sub, D), jnp.float32),
                       pltpu.VMEM((sub, D), jnp.float32),
                       pltpu.VMEM((sub, D), jnp.float32),
                       pltpu.SemaphoreType.DMA,
                       pltpu.SemaphoreType.DMA,
                       pltpu.SemaphoreType.DMA],
    )
    def k(ys_hbm, sa_hbm, sb_hbm, wa_hbm, wb_hbm, gsh_hbm, out_hbm,
          ia_v, ib_v, wa_v, wb_v, ya_v, yb_v, o_v, sem_a, sem_b, sem_g):
        wid = lax.axis_index("s") * NC + lax.axis_index("c")
        t0 = wid * n_per_w

        @pl.loop(0, nsub)
        def _(s):
            rows = pl.ds(t0 + s * sub, sub)
            pltpu.sync_copy(sa_hbm.at[rows], ia_v)
            pltpu.sync_copy(sb_hbm.at[rows], ib_v)
            pltpu.sync_copy(wa_hbm.at[rows], wa_v)
            pltpu.sync_copy(wb_hbm.at[rows], wb_v)
            ca = pltpu.async_copy(ys_hbm.at[ia_v], ya_v, sem_a)
            cb = pltpu.async_copy(ys_hbm.at[ib_v], yb_v, sem_b)
            cg = pltpu.async_copy(gsh_hbm.at[rows], o_v, sem_g)
            ca.wait()
            cb.wait()
            cg.wait()

            @pl.loop(0, sub)
            def _(r):
                rr = pl.ds(r, 1)
                wav = wa_v.at[rr, :][...]
                wbv = wb_v.at[rr, :][...]

                @pl.loop(0, D // 16, step=4)
                def _(c):
                    for j in range(4):
                        cc = pl.ds((c + j) * 16, 16)
                        o_v.at[rr, cc][...] += (
                            wav * ya_v.at[rr, cc][...]
                            + wbv * yb_v.at[rr, cc][...])

            pltpu.sync_copy(o_v, out_hbm.at[rows])

    return k(ys, sa, sb, wa, wb, gsh)


@jax.jit
def kernel(hidden_states, gate_w, shared_gate_w, Wgp, Wup, Wdn, Sgp, Sup, Sdn):
    x = hidden_states.reshape(T, D)

    full = lambda s: pl.BlockSpec(s, lambda *_: (0,) * len(s))
    tok = lambda d1: pl.BlockSpec((BT, d1), lambda t: (t, 0))

    sa, sb, wa, wb, bexp = pl.pallas_call(
        _route_body,
        in_specs=[full((T, D)), full((E, D))],
        out_specs=(full((T, 1)), full((T, 1)), full((T, 16)),
                   full((T, 16)), full((1, NBLK))),
        out_shape=(jax.ShapeDtypeStruct((T, 1), jnp.int32),
                   jax.ShapeDtypeStruct((T, 1), jnp.int32),
                   jax.ShapeDtypeStruct((T, 16), jnp.float32),
                   jax.ShapeDtypeStruct((T, 16), jnp.float32),
                   jax.ShapeDtypeStruct((1, NBLK), jnp.int32)),
    )(x, gate_w)

    gsh = pl.pallas_call(
        _shared_body,
        grid=(NT,),
        in_specs=[tok(D), full((1, D)), full((DFF, D)), full((DFF, D)),
                  full((D, DFF))],
        out_specs=tok(D),
        out_shape=jax.ShapeDtypeStruct((T, D), jnp.float32),
    )(x, shared_gate_w, Sgp, Sup, Sdn)

    idx = jnp.concatenate([sa.reshape(T), sb.reshape(T)]).reshape(-1, 32)
    xs = _dispatch(x, idx)

    ys = pl.pallas_call(
        _group_body,
        grid_spec=pltpu.PrefetchScalarGridSpec(
            num_scalar_prefetch=1,
            grid=(NBLK,),
            in_specs=[
                pl.BlockSpec((BTS, D), lambda i, bexp_ref: (i, 0)),
                pl.BlockSpec((1, DFF, D),
                             lambda i, be: (jnp.minimum(be[i], E - 1), 0, 0)),
                pl.BlockSpec((1, DFF, D),
                             lambda i, be: (jnp.minimum(be[i], E - 1), 0, 0)),
                pl.BlockSpec((1, D, DFF),
                             lambda i, be: (jnp.minimum(be[i], E - 1), 0, 0)),
            ],
            out_specs=pl.BlockSpec((BTS, D), lambda i, bexp_ref: (i, 0)),
        ),
        out_shape=jax.ShapeDtypeStruct((SPAD, D), jnp.float32),
    )(bexp.reshape(NBLK), xs, Wgp, Wup, Wdn)

    return _combine(ys, sa.reshape(T), sb.reshape(T), wa, wb, gsh)


# speedup vs baseline: 1.4248x; 1.0311x over previous
import functools
import math

import jax
from jax import lax
import jax.numpy as jnp
from jax.experimental import pallas as pl
from jax.experimental.pallas import tpu as pltpu
from jax.experimental.pallas import tpu_sc as plsc

T = 2048
D = 1024
E = 8
DFF = 1024
TOP_K = 2
_SCALE = 1.0 / math.sqrt(TOP_K)

BTS = 256
NBLK = 2 * T // BTS + E
SPAD = NBLK * BTS

NC, NS = 2, 16
NW = NC * NS

BT = 512
NT = T // BT


def _dot_t(a, b):
    return jax.lax.dot_general(a, b, (((1,), (1,)), ((), ())),
                               preferred_element_type=jnp.float32)


def _route_body(x_ref, gate_ref, sa_ref, sb_ref, wa_ref, wb_ref, bexp_ref):
    x = x_ref[...]
    logits = _dot_t(x, gate_ref[...])
    p = jax.nn.softmax(logits, axis=-1)
    m1 = jnp.max(p, axis=-1, keepdims=True)
    p_rest = jnp.where(p >= m1, -jnp.inf, p)
    m2 = jnp.max(p_rest, axis=-1, keepdims=True)
    mask = p >= m2
    pm = jnp.where(mask, p, 0.0)
    wd = pm / jnp.sum(pm, axis=-1, keepdims=True)

    ei = jax.lax.broadcasted_iota(jnp.int32, (T, E), 1)
    e1 = jnp.min(jnp.where(mask, ei, 8), axis=-1, keepdims=True)
    e2 = jnp.max(jnp.where(mask, ei, -1), axis=-1, keepdims=True)
    w1 = jnp.sum(jnp.where(ei == e1, wd, 0.0), axis=-1, keepdims=True)
    w2 = jnp.sum(jnp.where(ei == e2, wd, 0.0), axis=-1, keepdims=True)
    wa_ref[...] = (w1 * _SCALE) * jnp.ones((1, 16), jnp.float32)
    wb_ref[...] = (w2 * _SCALE) * jnp.ones((1, 16), jnp.float32)

    maskf = mask.astype(jnp.float32)
    s = maskf
    k = 1
    while k < T:
        s = s + jnp.concatenate([jnp.zeros((k, E), jnp.float32), s[:-k]],
                                axis=0)
        k *= 2
    rank = s - maskf
    count = jnp.sum(maskf, axis=0, keepdims=True)
    cpad = jnp.ceil(count * (1.0 / BTS)) * BTS
    base = jnp.zeros((1, 1), jnp.float32)
    bases = []
    for e in range(E):
        bases.append(base)
        base = base + cpad[:, e:e + 1]
    basev = jnp.concatenate(bases, axis=1)
    slotd = basev + rank
    sa = jnp.sum(jnp.where(ei == e1, slotd, 0.0), axis=-1, keepdims=True)
    sb = jnp.sum(jnp.where(ei == e2, slotd, 0.0), axis=-1, keepdims=True)
    sa_ref[...] = sa.astype(jnp.int32)
    sb_ref[...] = sb.astype(jnp.int32)

    ends = basev + cpad
    starts = jax.lax.broadcasted_iota(
        jnp.int32, (1, NBLK), 1).astype(jnp.float32) * BTS
    acc = jnp.zeros((1, NBLK), jnp.float32)
    for e in range(E):
        acc = acc + (starts >= ends[:, e:e + 1]).astype(jnp.float32)
    bexp_ref[...] = acc.astype(jnp.int32)


def _shared_body(x_ref, sgw_ref, sgp_ref, sup_ref, sdn_ref, gsh_ref):
    x = x_ref[...]
    gs = jax.nn.sigmoid(jnp.sum(x * sgw_ref[...], axis=1, keepdims=True))
    g = _dot_t(x, sgp_ref[...])
    u = _dot_t(x, sup_ref[...])
    h = g * jax.nn.sigmoid(g) * u
    gsh_ref[...] = (gs * _SCALE) * _dot_t(h, sdn_ref[...])


def _dispatch(x, idx2d):
    mesh = plsc.VectorSubcoreMesh(core_axis_name="c", subcore_axis_name="s")
    n_per_w = 2 * T // NW
    sub = 32
    nsub = n_per_w // sub

    @functools.partial(
        pl.kernel, mesh=mesh,
        out_type=jax.ShapeDtypeStruct((SPAD, D), jnp.float32),
        scratch_types=[pltpu.VMEM((nsub, sub), jnp.int32),
                       pltpu.VMEM((sub, D), jnp.float32),
                       pltpu.VMEM((sub, D), jnp.float32),
                       pltpu.VMEM((sub, D), jnp.float32),
                       pltpu.SemaphoreType.DMA,
                       pltpu.SemaphoreType.DMA,
                       pltpu.SemaphoreType.DMA,
                       pltpu.SemaphoreType.DMA,
                       pltpu.SemaphoreType.DMA,
                       pltpu.SemaphoreType.DMA],
    )
    def k(x_hbm, idx_hbm, xs_hbm, idx_v, rv0, rv1, rv2,
          ls0, ls1, ls2, ss0, ss1, ss2):
        wid = lax.axis_index("s") * NC + lax.axis_index("c")
        a0 = wid * n_per_w
        t0 = lax.rem(a0, T)
        pltpu.sync_copy(idx_hbm.at[pl.ds(wid * nsub, nsub)], idx_v)

        rvs, lss, sss = (rv0, rv1, rv2), (ls0, ls1, ls2), (ss0, ss1, ss2)
        loads = [pltpu.async_copy(x_hbm.at[pl.ds(t0 + s * sub, sub)],
                                  rvs[s], lss[s]) for s in range(3)]
        stores = {}
        for s in range(nsub):
            b = s % 3
            if s >= 3:
                stores[s - 3].wait()
                loads[b] = pltpu.async_copy(
                    x_hbm.at[pl.ds(t0 + s * sub, sub)], rvs[b], lss[b])
            loads[b].wait()
            stores[s] = pltpu.async_copy(rvs[b], xs_hbm.at[idx_v.at[s]],
                                         sss[b])
        for s in range(max(0, nsub - 3), nsub):
            stores[s].wait()

    return k(x, idx2d)


def _group_body(bexp_ref, xs_ref, wgp_ref, wup_ref, wdn_ref, ys_ref):
    i = pl.program_id(0)

    @pl.when(bexp_ref[i] < E)
    def _():
        x = xs_ref[...]
        g = _dot_t(x, wgp_ref[0])
        u = _dot_t(x, wup_ref[0])
        h = g * jax.nn.sigmoid(g) * u
        ys_ref[...] = _dot_t(h, wdn_ref[0])


def _combine(ys, sa, sb, wa, wb, gsh):
    mesh = plsc.VectorSubcoreMesh(core_axis_name="c", subcore_axis_name="s")
    n_per_w = T // NW
    sub = 32
    nsub = n_per_w // sub

    @functools.partial(
        pl.kernel, mesh=mesh,
        out_type=jax.ShapeDtypeStruct((T, D), jnp.float32),
        scratch_types=[pltpu.VMEM((sub,), jnp.int32),
                       pltpu.VMEM((sub,), jnp.int32),
                       pltpu.VMEM((sub, 16), jnp.float32),
                       pltpu.VMEM((sub, 16), jnp.float32),
                       pltpu.VMEM((sub, D), jnp.float32),
                       pltpu.VMEM((sub, D), jnp.float32),
                       pltpu.VMEM((sub, D), jnp.float32),
                       pltpu.SemaphoreType.DMA,
                       pltpu.SemaphoreType.DMA,
                       pltpu.SemaphoreType.DMA],
    )
    def k(ys_hbm, sa_hbm, sb_hbm, wa_hbm, wb_hbm, gsh_hbm, out_hbm,
          ia_v, ib_v, wa_v, wb_v, ya_v, yb_v, o_v, sem_a, sem_b, sem_g):
        wid = lax.axis_index("s") * NC + lax.axis_index("c")
        t0 = wid * n_per_w

        @pl.loop(0, nsub)
        def _(s):
            rows = pl.ds(t0 + s * sub, sub)
            pltpu.sync_copy(sa_hbm.at[rows], ia_v)
            pltpu.sync_copy(sb_hbm.at[rows], ib_v)
            pltpu.sync_copy(wa_hbm.at[rows], wa_v)
            pltpu.sync_copy(wb_hbm.at[rows], wb_v)
            ca = pltpu.async_copy(ys_hbm.at[ia_v], ya_v, sem_a)
            cb = pltpu.async_copy(ys_hbm.at[ib_v], yb_v, sem_b)
            cg = pltpu.async_copy(gsh_hbm.at[rows], o_v, sem_g)
            ca.wait()
            cb.wait()
            cg.wait()

            @pl.loop(0, sub)
            def _(r):
                rr = pl.ds(r, 1)
                wav = wa_v.at[rr, :][...]
                wbv = wb_v.at[rr, :][...]

                @pl.loop(0, D // 16, step=4)
                def _(c):
                    for j in range(4):
                        cc = pl.ds((c + j) * 16, 16)
                        o_v.at[rr, cc][...] += (
                            wav * ya_v.at[rr, cc][...]
                            + wbv * yb_v.at[rr, cc][...])

            pltpu.sync_copy(o_v, out_hbm.at[rows])

    return k(ys, sa, sb, wa, wb, gsh)


@jax.jit
def kernel(hidden_states, gate_w, shared_gate_w, Wgp, Wup, Wdn, Sgp, Sup, Sdn):
    x = hidden_states.reshape(T, D)

    full = lambda s: pl.BlockSpec(s, lambda *_: (0,) * len(s))
    tok = lambda d1: pl.BlockSpec((BT, d1), lambda t: (t, 0))

    sa, sb, wa, wb, bexp = pl.pallas_call(
        _route_body,
        in_specs=[full((T, D)), full((E, D))],
        out_specs=(full((T, 1)), full((T, 1)), full((T, 16)),
                   full((T, 16)), full((1, NBLK))),
        out_shape=(jax.ShapeDtypeStruct((T, 1), jnp.int32),
                   jax.ShapeDtypeStruct((T, 1), jnp.int32),
                   jax.ShapeDtypeStruct((T, 16), jnp.float32),
                   jax.ShapeDtypeStruct((T, 16), jnp.float32),
                   jax.ShapeDtypeStruct((1, NBLK), jnp.int32)),
    )(x, gate_w)

    gsh = pl.pallas_call(
        _shared_body,
        grid=(NT,),
        in_specs=[tok(D), full((1, D)), full((DFF, D)), full((DFF, D)),
                  full((D, DFF))],
        out_specs=tok(D),
        out_shape=jax.ShapeDtypeStruct((T, D), jnp.float32),
    )(x, shared_gate_w, Sgp, Sup, Sdn)

    idx = jnp.concatenate([sa.reshape(T), sb.reshape(T)]).reshape(-1, 32)
    xs = _dispatch(x, idx)

    ys = pl.pallas_call(
        _group_body,
        grid_spec=pltpu.PrefetchScalarGridSpec(
            num_scalar_prefetch=1,
            grid=(NBLK,),
            in_specs=[
                pl.BlockSpec((BTS, D), lambda i, bexp_ref: (i, 0)),
                pl.BlockSpec((1, DFF, D),
                             lambda i, be: (jnp.minimum(be[i], E - 1), 0, 0)),
                pl.BlockSpec((1, DFF, D),
                             lambda i, be: (jnp.minimum(be[i], E - 1), 0, 0)),
                pl.BlockSpec((1, D, DFF),
                             lambda i, be: (jnp.minimum(be[i], E - 1), 0, 0)),
            ],
            out_specs=pl.BlockSpec((BTS, D), lambda i, bexp_ref: (i, 0)),
        ),
        out_shape=jax.ShapeDtypeStruct((SPAD, D), jnp.float32),
    )(bexp.reshape(NBLK), xs, Wgp, Wup, Wdn)

    return _combine(ys, sa.reshape(T), sb.reshape(T), wa, wb, gsh)
